# Initial kernel scaffold; baseline (speedup 1.0000x reference)
import jax
import jax.numpy as jnp
from jax.experimental import pallas as pl


def kernel(moving, fixed):
    raise NotImplementedError("write your pallas kernel here")


# speedup vs baseline: 69.1692x; 69.1692x over previous
import functools

import jax
import jax.numpy as jnp
from jax import lax
from jax.experimental import pallas as pl
from jax.experimental.pallas import tpu as pltpu
from jax.experimental.pallas import tpu_sc as plsc

NBINS = 16
HIST = NBINS * NBINS + NBINS
NC = 2
NS = 16
NW = NC * NS


def _minmax_body(mov_ref, fix_ref, fmin_ref, fmax_ref, mmin_ref, mmax_ref):
    i = pl.program_id(1)
    m = mov_ref[0]
    f = fix_ref[0]
    shape = (1, 8, 128)
    pf_min = jnp.broadcast_to(jnp.min(f, axis=0)[None, None], shape)
    pf_max = jnp.broadcast_to(jnp.max(f, axis=0)[None, None], shape)
    pm_min = jnp.broadcast_to(jnp.min(m, axis=0)[None, None], shape)
    pm_max = jnp.broadcast_to(jnp.max(m, axis=0)[None, None], shape)

    @pl.when(i == 0)
    def _():
        fmin_ref[...] = pf_min
        fmax_ref[...] = pf_max
        mmin_ref[...] = pm_min
        mmax_ref[...] = pm_max

    @pl.when(i != 0)
    def _():
        fmin_ref[...] = jnp.minimum(fmin_ref[...], pf_min)
        fmax_ref[...] = jnp.maximum(fmax_ref[...], pf_max)
        mmin_ref[...] = jnp.minimum(mmin_ref[...], pm_min)
        mmax_ref[...] = jnp.maximum(mmax_ref[...], pm_max)


def _minmax(mov3, fix3, nblk):
    b, rows, _ = mov3.shape
    sub = rows // nblk
    spec = pl.BlockSpec((1, sub, 128), lambda bi, i: (bi, i, 0))
    ospec = pl.BlockSpec((1, 8, 128), lambda bi, i: (bi, 0, 0))
    out = jax.ShapeDtypeStruct((b, 8, 128), jnp.float32)
    return pl.pallas_call(
        _minmax_body,
        grid=(b, nblk),
        in_specs=[spec, spec],
        out_specs=[ospec, ospec, ospec, ospec],
        out_shape=[out, out, out, out],
    )(mov3, fix3)


def _terms_body(par_ref, mov_ref, fix_ref, ft_ref, mt_ref):
    bi = pl.program_id(0)
    fbin = par_ref[bi, 0]
    foff = par_ref[bi, 1]
    mbin = par_ref[bi, 2]
    moff = par_ref[bi, 3]
    ft_ref[...] = fix_ref[...] / fbin - foff
    mt_ref[...] = mov_ref[...] / mbin - moff


def _terms(mov3, fix3, params, nblk):
    b, rows, _ = mov3.shape
    sub = rows // nblk
    spec = pl.BlockSpec((1, sub, 128), lambda bi, i: (bi, i, 0))
    pspec = pl.BlockSpec((b, 4), lambda bi, i: (0, 0), memory_space=pltpu.SMEM)
    out = jax.ShapeDtypeStruct((b, rows, 128), jnp.float32)
    return pl.pallas_call(
        _terms_body,
        grid=(b, nblk),
        in_specs=[pspec, spec, spec],
        out_specs=[spec, spec],
        out_shape=[out, out],
    )(params, mov3, fix3)


def _make_sc_hist(n_total, batches):
    tiles_per_b = NW // batches
    vpt = n_total // tiles_per_b
    chunk = 16384
    nchunk = vpt // chunk
    steps = chunk // 16
    mesh = plsc.VectorSubcoreMesh(core_axis_name="c", subcore_axis_name="s")

    @functools.partial(
        pl.kernel,
        mesh=mesh,
        out_type=jax.ShapeDtypeStruct((NW, HIST), jnp.float32),
        compiler_params=pltpu.CompilerParams(needs_layout_passes=False),
        scratch_types=[
            pltpu.VMEM((chunk,), jnp.float32),
            pltpu.VMEM((chunk,), jnp.float32),
            pltpu.VMEM((16 * HIST,), jnp.float32),
            pltpu.VMEM((HIST,), jnp.float32),
        ],
    )
    def sc_hist(ft_hbm, mt_hbm, out_hbm, mbuf, fbuf, hist, hsum):
        cid = lax.axis_index("c")
        sid = lax.axis_index("s")
        wid = sid * NC + cid
        batch = wid // tiles_per_b
        slot = wid % tiles_per_b
        base = slot * vpt

        zero = jnp.zeros((16,), jnp.float32)
        for k in range(16 * HIST // 16):
            hist[pl.ds(k * 16, 16)] = zero

        lanebase = lax.iota(jnp.int32, 16) * HIST
        ones = jnp.ones((16,), jnp.float32)

        def chunk_body(ci, carry):
            start = base + ci * chunk
            pltpu.sync_copy(mt_hbm.at[batch, pl.ds(start, chunk)], mbuf)
            pltpu.sync_copy(ft_hbm.at[batch, pl.ds(start, chunk)], fbuf)

            def step(si, c2):
                o = si * 16
                f_term = fbuf[pl.ds(o, 16)]
                m_term = mbuf[pl.ds(o, 16)]
                f_ix = jnp.clip(f_term.astype(jnp.int32), 2, NBINS - 3)
                m_ix = jnp.clip(m_term.astype(jnp.int32), 2, NBINS - 3)
                flat = lanebase + f_ix * NBINS + m_ix
                sixth = jnp.float32(1.0 / 6.0)
                for off in (-1, 0, 1, 2):
                    u = (m_ix + off).astype(jnp.float32) - m_term
                    au = jnp.abs(u)
                    su = au * au
                    r1 = (4.0 - 6.0 * su + 3.0 * su * au) * sixth
                    r2 = (8.0 - 12.0 * au + 6.0 * su - su * au) * sixth
                    w = jnp.where(au < 1.0, r1,
                                  jnp.where(au < 2.0, r2, jnp.zeros_like(u)))
                    plsc.addupdate_scatter(hist, [flat + off], w)
                cidx = lanebase + (f_ix + NBINS * NBINS)
                plsc.addupdate_scatter(hist, [cidx], ones)
                return c2

            lax.fori_loop(0, steps, step, 0)
            return carry

        lax.fori_loop(0, nchunk, chunk_body, 0)

        for k in range(HIST // 16):
            acc = hist[pl.ds(k * 16, 16)]
            for r in range(1, 16):
                acc = acc + hist[pl.ds(r * HIST + k * 16, 16)]
            hsum[pl.ds(k * 16, 16)] = acc

        pltpu.sync_copy(hsum, out_hbm.at[wid])

    return sc_hist


def _final_body(h_ref, out_ref, *, n_total):
    x = h_ref[...]
    s = jnp.sum(x, axis=1)
    joint = s[:, :NBINS, :]
    counts = jnp.sum(s[:, NBINS:, :], axis=1)
    jsum = jnp.sum(joint, axis=(1, 2), keepdims=True)
    joint_norm = joint / jsum
    fixed_pdf = counts / float(n_total)
    moving_pdf = jnp.sum(joint_norm, axis=1)

    def plogp(p):
        safe = jnp.where(p > 0, p, jnp.ones_like(p))
        return jnp.where(p > 0, p * jnp.log(safe), jnp.zeros_like(p))

    mi = (jnp.sum(plogp(joint_norm), axis=(1, 2))
          - jnp.sum(plogp(moving_pdf), axis=1)
          - jnp.sum(plogp(fixed_pdf), axis=1))
    out_ref[...] = jnp.full((1, 1), -1.0, jnp.float32) * jnp.mean(mi)


def _final(hists4d, n_total):
    return pl.pallas_call(
        functools.partial(_final_body, n_total=n_total),
        out_shape=jax.ShapeDtypeStruct((1, 1), jnp.float32),
    )(hists4d)


def kernel(moving, fixed):
    b = moving.shape[0]
    mov = moving.reshape(b, -1)
    fix = fixed.reshape(b, -1)
    n = mov.shape[1]

    mov3 = mov.reshape(b, n // 128, 128)
    fix3 = fix.reshape(b, n // 128, 128)
    fmin128, fmax128, mmin128, mmax128 = _minmax(mov3, fix3, 8)
    fmin = jnp.min(fmin128, axis=(1, 2))
    fmax = jnp.max(fmax128, axis=(1, 2))
    mmin = jnp.min(mmin128, axis=(1, 2))
    mmax = jnp.max(mmax128, axis=(1, 2))

    fbin = (fmax - fmin)[:, None] / (float(NBINS) - 4.0)
    mbin = (mmax - mmin)[:, None] / (float(NBINS) - 4.0)
    f_term = fix / fbin - (fmin[:, None] / fbin - 2.0)
    m_term = mov / mbin - (mmin[:, None] / mbin - 2.0)

    sc_hist = _make_sc_hist(n, b)
    hists = sc_hist(f_term, m_term)

    hists4d = hists.reshape(b, NW // b, NBINS + 1, NBINS)
    s = jnp.sum(hists4d, axis=1)
    joint = s[:, :NBINS, :]
    counts = s[:, NBINS, :]
    joint_norm = joint / jnp.sum(joint, axis=(1, 2), keepdims=True)
    fixed_pdf = counts / float(n)
    moving_pdf = jnp.sum(joint_norm, axis=1)

    def plogp(p):
        safe = jnp.where(p > 0, p, jnp.ones_like(p))
        return jnp.where(p > 0, p * jnp.log(safe), jnp.zeros_like(p))

    mi = (jnp.sum(plogp(joint_norm), axis=(1, 2))
          - jnp.sum(plogp(moving_pdf), axis=1)
          - jnp.sum(plogp(fixed_pdf), axis=1))
    return -1.0 * jnp.mean(mi)


# speedup vs baseline: 69.1985x; 1.0004x over previous
import functools

import jax
import jax.numpy as jnp
from jax import lax
from jax.experimental import pallas as pl
from jax.experimental.pallas import tpu as pltpu
from jax.experimental.pallas import tpu_sc as plsc

NBINS = 16
HIST = NBINS * NBINS + NBINS
NC = 2
NS = 16
NW = NC * NS


def _minmax_body(mov_ref, fix_ref, fmin_ref, fmax_ref, mmin_ref, mmax_ref):
    i = pl.program_id(1)
    m = mov_ref[0]
    f = fix_ref[0]
    shape = (1, 8, 128)
    pf_min = jnp.broadcast_to(jnp.min(f, axis=0)[None, None], shape)
    pf_max = jnp.broadcast_to(jnp.max(f, axis=0)[None, None], shape)
    pm_min = jnp.broadcast_to(jnp.min(m, axis=0)[None, None], shape)
    pm_max = jnp.broadcast_to(jnp.max(m, axis=0)[None, None], shape)

    @pl.when(i == 0)
    def _():
        fmin_ref[...] = pf_min
        fmax_ref[...] = pf_max
        mmin_ref[...] = pm_min
        mmax_ref[...] = pm_max

    @pl.when(i != 0)
    def _():
        fmin_ref[...] = jnp.minimum(fmin_ref[...], pf_min)
        fmax_ref[...] = jnp.maximum(fmax_ref[...], pf_max)
        mmin_ref[...] = jnp.minimum(mmin_ref[...], pm_min)
        mmax_ref[...] = jnp.maximum(mmax_ref[...], pm_max)


def _minmax(mov3, fix3, nblk):
    b, rows, _ = mov3.shape
    sub = rows // nblk
    spec = pl.BlockSpec((1, sub, 128), lambda bi, i: (bi, i, 0))
    ospec = pl.BlockSpec((1, 8, 128), lambda bi, i: (bi, 0, 0))
    out = jax.ShapeDtypeStruct((b, 8, 128), jnp.float32)
    return pl.pallas_call(
        _minmax_body,
        grid=(b, nblk),
        in_specs=[spec, spec],
        out_specs=[ospec, ospec, ospec, ospec],
        out_shape=[out, out, out, out],
    )(mov3, fix3)


def _make_sc_hist(n_total, batches):
    tiles_per_b = NW // batches
    vpt = n_total // tiles_per_b
    chunk = 16384
    nchunk = vpt // chunk
    steps = chunk // 16
    mesh = plsc.VectorSubcoreMesh(core_axis_name="c", subcore_axis_name="s")

    @functools.partial(
        pl.kernel,
        mesh=mesh,
        out_type=jax.ShapeDtypeStruct((NW, HIST), jnp.float32),
        compiler_params=pltpu.CompilerParams(needs_layout_passes=False),
        scratch_types=[
            pltpu.VMEM((chunk,), jnp.float32),
            pltpu.VMEM((chunk,), jnp.float32),
            pltpu.VMEM((16 * HIST,), jnp.float32),
            pltpu.VMEM((HIST,), jnp.float32),
        ],
    )
    def sc_hist(ft_hbm, mt_hbm, out_hbm, mbuf, fbuf, hist, hsum):
        cid = lax.axis_index("c")
        sid = lax.axis_index("s")
        wid = sid * NC + cid
        batch = wid // tiles_per_b
        slot = wid % tiles_per_b
        base = slot * vpt

        zero = jnp.zeros((16,), jnp.float32)
        for k in range(16 * HIST // 16):
            hist[pl.ds(k * 16, 16)] = zero

        lanebase = lax.iota(jnp.int32, 16) * HIST
        ones = jnp.ones((16,), jnp.float32)

        def chunk_body(ci, carry):
            start = base + ci * chunk
            pltpu.sync_copy(mt_hbm.at[batch, pl.ds(start, chunk)], mbuf)
            pltpu.sync_copy(ft_hbm.at[batch, pl.ds(start, chunk)], fbuf)

            def step(si, c2):
                o = si * 16
                f_term = fbuf[pl.ds(o, 16)]
                m_term = mbuf[pl.ds(o, 16)]
                f_ix = jnp.clip(f_term.astype(jnp.int32), 2, NBINS - 3)
                m_ix = jnp.clip(m_term.astype(jnp.int32), 2, NBINS - 3)
                flat = lanebase + f_ix * NBINS + m_ix
                sixth = jnp.float32(1.0 / 6.0)
                for off in (-1, 0, 1, 2):
                    u = (m_ix + off).astype(jnp.float32) - m_term
                    au = jnp.abs(u)
                    su = au * au
                    r1 = (4.0 - 6.0 * su + 3.0 * su * au) * sixth
                    r2 = (8.0 - 12.0 * au + 6.0 * su - su * au) * sixth
                    w = jnp.where(au < 1.0, r1,
                                  jnp.where(au < 2.0, r2, jnp.zeros_like(u)))
                    plsc.addupdate_scatter(hist, [flat + off], w)
                cidx = lanebase + (f_ix + NBINS * NBINS)
                plsc.addupdate_scatter(hist, [cidx], ones)
                return c2

            lax.fori_loop(0, steps, step, 0)
            return carry

        lax.fori_loop(0, nchunk, chunk_body, 0)

        for k in range(HIST // 16):
            acc = hist[pl.ds(k * 16, 16)]
            for r in range(1, 16):
                acc = acc + hist[pl.ds(r * HIST + k * 16, 16)]
            hsum[pl.ds(k * 16, 16)] = acc

        pltpu.sync_copy(hsum, out_hbm.at[wid])

    return sc_hist


def kernel(moving, fixed):
    b = moving.shape[0]
    mov = moving.reshape(b, -1)
    fix = fixed.reshape(b, -1)
    n = mov.shape[1]

    mov3 = mov.reshape(b, n // 128, 128)
    fix3 = fix.reshape(b, n // 128, 128)
    fmin128, fmax128, mmin128, mmax128 = _minmax(mov3, fix3, 8)
    fmin = jnp.min(fmin128, axis=(1, 2))
    fmax = jnp.max(fmax128, axis=(1, 2))
    mmin = jnp.min(mmin128, axis=(1, 2))
    mmax = jnp.max(mmax128, axis=(1, 2))

    fbin = (fmax - fmin)[:, None] / (float(NBINS) - 4.0)
    mbin = (mmax - mmin)[:, None] / (float(NBINS) - 4.0)
    f_term = fix / fbin - (fmin[:, None] / fbin - 2.0)
    m_term = mov / mbin - (mmin[:, None] / mbin - 2.0)

    sc_hist = _make_sc_hist(n, b)
    hists = sc_hist(f_term, m_term)

    hists4d = hists.reshape(b, NW // b, NBINS + 1, NBINS)
    s = jnp.sum(hists4d, axis=1)
    joint = s[:, :NBINS, :]
    counts = s[:, NBINS, :]
    joint_norm = joint / jnp.sum(joint, axis=(1, 2), keepdims=True)
    fixed_pdf = counts / float(n)
    moving_pdf = jnp.sum(joint_norm, axis=1)

    def plogp(p):
        safe = jnp.where(p > 0, p, jnp.ones_like(p))
        return jnp.where(p > 0, p * jnp.log(safe), jnp.zeros_like(p))

    mi = (jnp.sum(plogp(joint_norm), axis=(1, 2))
          - jnp.sum(plogp(moving_pdf), axis=1)
          - jnp.sum(plogp(fixed_pdf), axis=1))
    return -1.0 * jnp.mean(mi)


# speedup vs baseline: 71.8689x; 1.0386x over previous
import functools

import jax
import jax.numpy as jnp
from jax import lax
from jax.experimental import pallas as pl
from jax.experimental.pallas import tpu as pltpu
from jax.experimental.pallas import tpu_sc as plsc

NBINS = 16
HIST = NBINS * NBINS + NBINS
NC = 2
NS = 16
NW = NC * NS


def _minmax_body(mov_ref, fix_ref, fmin_ref, fmax_ref, mmin_ref, mmax_ref):
    i = pl.program_id(1)
    m = mov_ref[0]
    f = fix_ref[0]
    shape = (1, 8, 128)
    pf_min = jnp.broadcast_to(jnp.min(f, axis=0)[None, None], shape)
    pf_max = jnp.broadcast_to(jnp.max(f, axis=0)[None, None], shape)
    pm_min = jnp.broadcast_to(jnp.min(m, axis=0)[None, None], shape)
    pm_max = jnp.broadcast_to(jnp.max(m, axis=0)[None, None], shape)

    @pl.when(i == 0)
    def _():
        fmin_ref[...] = pf_min
        fmax_ref[...] = pf_max
        mmin_ref[...] = pm_min
        mmax_ref[...] = pm_max

    @pl.when(i != 0)
    def _():
        fmin_ref[...] = jnp.minimum(fmin_ref[...], pf_min)
        fmax_ref[...] = jnp.maximum(fmax_ref[...], pf_max)
        mmin_ref[...] = jnp.minimum(mmin_ref[...], pm_min)
        mmax_ref[...] = jnp.maximum(mmax_ref[...], pm_max)


def _minmax(mov3, fix3, nblk):
    b, rows, _ = mov3.shape
    sub = rows // nblk
    spec = pl.BlockSpec((1, sub, 128), lambda bi, i: (bi, i, 0))
    ospec = pl.BlockSpec((1, 8, 128), lambda bi, i: (bi, 0, 0))
    out = jax.ShapeDtypeStruct((b, 8, 128), jnp.float32)
    return pl.pallas_call(
        _minmax_body,
        grid=(b, nblk),
        in_specs=[spec, spec],
        out_specs=[ospec, ospec, ospec, ospec],
        out_shape=[out, out, out, out],
    )(mov3, fix3)


def _make_sc_hist(n_total, batches):
    tiles_per_b = NW // batches
    vpt = n_total // tiles_per_b
    chunk = 16384
    nchunk = vpt // chunk
    steps = chunk // 16
    mesh = plsc.VectorSubcoreMesh(core_axis_name="c", subcore_axis_name="s")

    @functools.partial(
        pl.kernel,
        mesh=mesh,
        out_type=jax.ShapeDtypeStruct((NW, HIST), jnp.float32),
        compiler_params=pltpu.CompilerParams(needs_layout_passes=False),
        scratch_types=[
            pltpu.VMEM((2, chunk), jnp.float32),
            pltpu.VMEM((2, chunk), jnp.float32),
            pltpu.VMEM((16 * HIST,), jnp.float32),
            pltpu.VMEM((HIST,), jnp.float32),
            pltpu.SemaphoreType.DMA,
            pltpu.SemaphoreType.DMA,
        ],
    )
    def sc_hist(ft_hbm, mt_hbm, out_hbm, mbuf, fbuf, hist, hsum, sem0, sem1):
        cid = lax.axis_index("c")
        sid = lax.axis_index("s")
        wid = sid * NC + cid
        batch = wid // tiles_per_b
        slot = wid % tiles_per_b
        base = slot * vpt

        zero = jnp.zeros((16,), jnp.float32)
        for k in range(16 * HIST // 16):
            hist[pl.ds(k * 16, 16)] = zero

        lanebase = lax.iota(jnp.int32, 16) * HIST
        ones = jnp.ones((16,), jnp.float32)

        def one_step(buf_sel, si):
            o = si * 16
            f_term = fbuf[buf_sel, pl.ds(o, 16)]
            m_term = mbuf[buf_sel, pl.ds(o, 16)]
            f_ix = jnp.clip(f_term.astype(jnp.int32), 2, NBINS - 3)
            m_ix = jnp.clip(m_term.astype(jnp.int32), 2, NBINS - 3)
            flat = lanebase + f_ix * NBINS + m_ix
            sixth = jnp.float32(1.0 / 6.0)
            for off in (-1, 0, 1, 2):
                u = (m_ix + off).astype(jnp.float32) - m_term
                au = jnp.abs(u)
                su = au * au
                r1 = (4.0 - 6.0 * su + 3.0 * su * au) * sixth
                r2 = (8.0 - 12.0 * au + 6.0 * su - su * au) * sixth
                w = jnp.where(au < 1.0, r1,
                              jnp.where(au < 2.0, r2, jnp.zeros_like(u)))
                plsc.addupdate_scatter(hist, [flat + off], w)
            cidx = lanebase + (f_ix + NBINS * NBINS)
            plsc.addupdate_scatter(hist, [cidx], ones)

        def start_fetch(buf_sel, ci):
            start = base + ci * chunk
            cm = pltpu.async_copy(mt_hbm.at[batch, pl.ds(start, chunk)],
                                  mbuf.at[buf_sel], sem0)
            cf = pltpu.async_copy(ft_hbm.at[batch, pl.ds(start, chunk)],
                                  fbuf.at[buf_sel], sem1)
            return cm, cf

        unroll = 4
        cm, cf = start_fetch(0, 0)
        for ci in range(nchunk):
            sel = ci % 2
            cm.wait()
            cf.wait()
            if ci + 1 < nchunk:
                cm, cf = start_fetch((ci + 1) % 2, ci + 1)

            def step(sg, c2):
                for j in range(unroll):
                    one_step(sel, sg * unroll + j)
                return c2

            lax.fori_loop(0, steps // unroll, step, 0)

        for k in range(HIST // 16):
            acc = hist[pl.ds(k * 16, 16)]
            for r in range(1, 16):
                acc = acc + hist[pl.ds(r * HIST + k * 16, 16)]
            hsum[pl.ds(k * 16, 16)] = acc

        pltpu.sync_copy(hsum, out_hbm.at[wid])

    return sc_hist


def kernel(moving, fixed):
    b = moving.shape[0]
    mov = moving.reshape(b, -1)
    fix = fixed.reshape(b, -1)
    n = mov.shape[1]

    mov3 = mov.reshape(b, n // 128, 128)
    fix3 = fix.reshape(b, n // 128, 128)
    fmin128, fmax128, mmin128, mmax128 = _minmax(mov3, fix3, 8)
    fmin = jnp.min(fmin128, axis=(1, 2))
    fmax = jnp.max(fmax128, axis=(1, 2))
    mmin = jnp.min(mmin128, axis=(1, 2))
    mmax = jnp.max(mmax128, axis=(1, 2))

    fbin = (fmax - fmin)[:, None] / (float(NBINS) - 4.0)
    mbin = (mmax - mmin)[:, None] / (float(NBINS) - 4.0)
    f_term = fix / fbin - (fmin[:, None] / fbin - 2.0)
    m_term = mov / mbin - (mmin[:, None] / mbin - 2.0)

    sc_hist = _make_sc_hist(n, b)
    hists = sc_hist(f_term, m_term)

    hists4d = hists.reshape(b, NW // b, NBINS + 1, NBINS)
    s = jnp.sum(hists4d, axis=1)
    joint = s[:, :NBINS, :]
    counts = s[:, NBINS, :]
    joint_norm = joint / jnp.sum(joint, axis=(1, 2), keepdims=True)
    fixed_pdf = counts / float(n)
    moving_pdf = jnp.sum(joint_norm, axis=1)

    def plogp(p):
        safe = jnp.where(p > 0, p, jnp.ones_like(p))
        return jnp.where(p > 0, p * jnp.log(safe), jnp.zeros_like(p))

    mi = (jnp.sum(plogp(joint_norm), axis=(1, 2))
          - jnp.sum(plogp(moving_pdf), axis=1)
          - jnp.sum(plogp(fixed_pdf), axis=1))
    return -1.0 * jnp.mean(mi)
